# baseline (speedup 1.0000x reference)
import functools

import jax
import jax.numpy as jnp
from jax import lax
from jax.experimental import pallas as pl
from jax.experimental.pallas import tpu as pltpu
from jax.experimental.pallas import tpu_sc as plsc

VOCAB = 100000
HIDDEN = 768
BATCH = 4
SEQ = 8192

NC, NS, L = 2, 16, 16
NW = NC * NS
SEQ_PER_W = SEQ // NW
CS = 32
NCHUNK = SEQ_PER_W // CS
HV = HIDDEN // L


def _embed_body(ids_hbm, tok_hbm, pos_hbm, pidx_hbm, out_hbm,
                idx_v, pidx_v, tok_v, pos_v, sem):
    wid = lax.axis_index("s") * NC + lax.axis_index("c")
    base = wid * SEQ_PER_W

    pltpu.sync_copy(pidx_hbm.at[pl.ds(base, SEQ_PER_W)], pidx_v)
    for b in range(BATCH):
        pltpu.sync_copy(ids_hbm.at[b, pl.ds(base, SEQ_PER_W)], idx_v.at[b])

    def chunk_body(c, carry):
        off = c * CS
        cps = [pltpu.async_copy(pos_hbm.at[pidx_v.at[pl.ds(off, CS)]],
                                pos_v, sem)]
        for b in range(BATCH):
            cps.append(pltpu.async_copy(tok_hbm.at[idx_v.at[b, pl.ds(off, CS)]],
                                        tok_v.at[b], sem))
        for cp in cps:
            cp.wait()

        def row_body(r, rcarry):
            for b in range(BATCH):
                for h in range(HV):
                    sl = pl.ds(h * L, L)
                    tok_v[b, r, sl] = tok_v[b, r, sl] + pos_v[r, sl]
            return rcarry
        lax.fori_loop(0, CS, row_body, 0)

        for b in range(BATCH):
            pltpu.sync_copy(tok_v.at[b], out_hbm.at[b, pl.ds(base + off, CS)])
        return carry

    lax.fori_loop(0, NCHUNK, chunk_body, 0)


def kernel(input_ids, tok_embed, pos_embed, past_seq_len=0):
    ids32 = input_ids.astype(jnp.int32)
    pos_idx = (jnp.asarray(past_seq_len, jnp.int32)
               + jnp.arange(SEQ, dtype=jnp.int32))

    mesh = plsc.VectorSubcoreMesh(core_axis_name="c", subcore_axis_name="s")
    run = pl.kernel(
        _embed_body,
        out_type=jax.ShapeDtypeStruct((BATCH, SEQ, HIDDEN), jnp.float32),
        mesh=mesh,
        scratch_types=[
            pltpu.VMEM((BATCH, SEQ_PER_W), jnp.int32),
            pltpu.VMEM((SEQ_PER_W,), jnp.int32),
            pltpu.VMEM((BATCH, CS, HIDDEN), jnp.float32),
            pltpu.VMEM((CS, HIDDEN), jnp.float32),
            pltpu.SemaphoreType.DMA,
        ],
    )
    return run(ids32, tok_embed, pos_embed, pos_idx)


# speedup vs baseline: 2.0705x; 2.0705x over previous
import jax
import jax.numpy as jnp
from jax import lax
from jax.experimental import pallas as pl
from jax.experimental.pallas import tpu as pltpu
from jax.experimental.pallas import tpu_sc as plsc

VOCAB = 100000
HIDDEN = 768
BATCH = 4
SEQ = 8192

NC, NS, L = 2, 16, 16
NW = NC * NS
SEQ_PER_W = SEQ // NW
CS = 8
NCHUNK = SEQ_PER_W // CS
NB = 4
HV = HIDDEN // L


def _embed_body(ids_hbm, tok_hbm, pos_hbm, pidx_hbm, out_hbm,
                idx_v, pidx_v, tok_v, pos_v, gsem, ssem):
    wid = lax.axis_index("s") * NC + lax.axis_index("c")
    base = wid * SEQ_PER_W

    pltpu.sync_copy(pidx_hbm.at[pl.ds(base, SEQ_PER_W)], pidx_v)
    for b in range(BATCH):
        pltpu.sync_copy(ids_hbm.at[b, pl.ds(base, SEQ_PER_W)], idx_v.at[b])

    def fire_gather(c, k):
        off = c * CS
        pltpu.async_copy(pos_hbm.at[pidx_v.at[pl.ds(off, CS)]],
                         pos_v.at[k], gsem.at[k])
        for b in range(BATCH):
            pltpu.async_copy(tok_hbm.at[idx_v.at[b, pl.ds(off, CS)]],
                             tok_v.at[k, b], gsem.at[k])

    def wait_gather(k):
        pltpu.make_async_copy(pos_hbm.at[pl.ds(0, CS)],
                              pos_v.at[k], gsem.at[k]).wait()
        for b in range(BATCH):
            pltpu.make_async_copy(tok_hbm.at[pl.ds(0, CS)],
                                  tok_v.at[k, b], gsem.at[k]).wait()

    def fire_store(c, k):
        off = base + c * CS
        for b in range(BATCH):
            pltpu.async_copy(tok_v.at[k, b],
                             out_hbm.at[b, pl.ds(off, CS)], ssem.at[k])

    def wait_store(k):
        for b in range(BATCH):
            pltpu.make_async_copy(tok_v.at[k, b],
                                  out_hbm.at[b, pl.ds(0, CS)], ssem.at[k]).wait()

    def add_slot(k):
        def row_body(r, rc):
            for h in range(HV):
                sl = pl.ds(h * L, L)
                p = pos_v[k, r, sl]
                for b in range(BATCH):
                    tok_v[k, b, r, sl] = tok_v[k, b, r, sl] + p
            return rc
        lax.fori_loop(0, CS, row_body, 0)

    fire_gather(0, 0)
    fire_gather(1, 1)

    def outer(t, carry):
        for ks in range(NB):
            c = t * NB + ks
            k2 = (ks + 2) % NB

            @pl.when(c >= 2)
            def _():
                wait_store(k2)

            @pl.when(c + 2 < NCHUNK)
            def _():
                fire_gather(c + 2, k2)

            wait_gather(ks)
            add_slot(ks)
            fire_store(c, ks)
        return carry

    lax.fori_loop(0, NCHUNK // NB, outer, 0)

    wait_store((NCHUNK - 2) % NB)
    wait_store((NCHUNK - 1) % NB)


def kernel(input_ids, tok_embed, pos_embed, past_seq_len=0):
    ids32 = input_ids.astype(jnp.int32)
    pos_idx = (jnp.asarray(past_seq_len, jnp.int32)
               + jnp.arange(SEQ, dtype=jnp.int32))

    mesh = plsc.VectorSubcoreMesh(core_axis_name="c", subcore_axis_name="s")
    run = pl.kernel(
        _embed_body,
        out_type=jax.ShapeDtypeStruct((BATCH, SEQ, HIDDEN), jnp.float32),
        mesh=mesh,
        scratch_types=[
            pltpu.VMEM((BATCH, SEQ_PER_W), jnp.int32),
            pltpu.VMEM((SEQ_PER_W,), jnp.int32),
            pltpu.VMEM((NB, BATCH, CS, HIDDEN), jnp.float32),
            pltpu.VMEM((NB, CS, HIDDEN), jnp.float32),
            pltpu.SemaphoreType.DMA((NB,)),
            pltpu.SemaphoreType.DMA((NB,)),
        ],
    )
    return run(ids32, tok_embed, pos_embed, pos_idx)
